# Initial kernel scaffold; baseline (speedup 1.0000x reference)
import functools

import jax
import jax.numpy as jnp
from jax import lax
from jax.experimental import pallas as pl
from jax.experimental.pallas import tpu as pltpu
from jax.experimental.pallas import tpu_sc as plsc

_INFO = plsc.get_sparse_core_info()
_NC = _INFO.num_cores
_NS = _INFO.num_subcores
_NW = _NC * _NS

_CB = 16


@functools.lru_cache(maxsize=None)
def _make_sc_kernel(B, L, D, V):
    del V
    per_w = B // _NW
    n_chunks = per_w // _CB
    rows_per_chunk = _CB * L
    nk = D // 16

    mesh = plsc.VectorSubcoreMesh(core_axis_name="c", subcore_axis_name="s")

    @functools.partial(
        pl.kernel,
        mesh=mesh,
        out_type=jax.ShapeDtypeStruct((B * L,), jnp.float32),
        scratch_types=[
            pltpu.VMEM((_CB, L), jnp.int32),
            pltpu.VMEM((_CB,), jnp.int32),
            pltpu.VMEM((_CB, D), jnp.float32),
            pltpu.VMEM((rows_per_chunk, D), jnp.float32),
            pltpu.VMEM((rows_per_chunk,), jnp.float32),
            pltpu.SemaphoreType.DMA,
        ],
    )
    def sc_kernel(ctx_hbm, center_hbm, ev_hbm, eu_hbm, out_hbm,
                  ctx_v, cen_v, vrows_v, urows_v, out_v, sem):
        wid = lax.axis_index("s") * _NC + lax.axis_index("c")
        wbase = wid * per_w

        def chunk_body(ci, carry):
            gbase = pl.multiple_of(wbase + ci * _CB, _CB)
            pltpu.sync_copy(center_hbm.at[pl.ds(gbase, _CB)], cen_v)
            pltpu.sync_copy(ctx_hbm.at[pl.ds(gbase, _CB)], ctx_v)
            copies = [pltpu.async_copy(ev_hbm.at[cen_v], vrows_v, sem)]
            for b in range(_CB):
                copies.append(pltpu.async_copy(
                    eu_hbm.at[ctx_v.at[b]],
                    urows_v.at[pl.ds(b * L, L)], sem))
            for cp in copies:
                cp.wait()
            for b in range(_CB):
                vk = [vrows_v[b, pl.ds(k * 16, 16)] for k in range(nk)]

                def l_body(l, c, b=b, vk=vk):
                    row = b * L + l
                    acc = urows_v[row, pl.ds(0, 16)] * vk[0]
                    for k in range(1, nk):
                        acc = acc + urows_v[row, pl.ds(k * 16, 16)] * vk[k]
                    out_v[row] = jnp.sum(acc)
                    return c

                lax.fori_loop(0, L, l_body, 0)
            pltpu.sync_copy(
                out_v, out_hbm.at[pl.ds(gbase * L, rows_per_chunk)])
            return carry

        lax.fori_loop(0, n_chunks, chunk_body, 0)

    return sc_kernel


def kernel(center, contexts_and_negatives, embed_v, embed_u):
    B, L = contexts_and_negatives.shape
    V, D = embed_v.shape
    center_flat = center.reshape(B).astype(jnp.int32)
    ctx = contexts_and_negatives.astype(jnp.int32)
    out = _make_sc_kernel(B, L, D, V)(ctx, center_flat, embed_v, embed_u)
    return out.reshape(B, 1, L)


# speedup vs baseline: 10.7569x; 10.7569x over previous
import functools

import jax
import jax.numpy as jnp
from jax import lax
from jax.experimental import pallas as pl
from jax.experimental.pallas import tpu as pltpu
from jax.experimental.pallas import tpu_sc as plsc

_INFO = plsc.get_sparse_core_info()
_NC = _INFO.num_cores
_NS = _INFO.num_subcores
_NW = _NC * _NS

_CB = 16


@functools.lru_cache(maxsize=None)
def _make_sc_kernel(B, L, D, V):
    del V
    per_w = B // _NW
    n_chunks = per_w // _CB
    rows_per_chunk = _CB * L
    nk = D // 16

    mesh = plsc.VectorSubcoreMesh(core_axis_name="c", subcore_axis_name="s")

    @functools.partial(
        pl.kernel,
        mesh=mesh,
        compiler_params=pltpu.CompilerParams(use_tc_tiling_on_sc=False),
        out_type=jax.ShapeDtypeStruct((B * L,), jnp.float32),
        scratch_types=[
            pltpu.VMEM((_CB, L), jnp.int32),
            pltpu.VMEM((_CB,), jnp.int32),
            pltpu.VMEM((_CB, D), jnp.float32),
            pltpu.VMEM((rows_per_chunk, D), jnp.float32),
            pltpu.VMEM((rows_per_chunk + 16,), jnp.float32),
            pltpu.SemaphoreType.DMA,
        ],
    )
    def sc_kernel(ctx_hbm, center_hbm, ev_hbm, eu_hbm, out_hbm,
                  ctx_v, cen_v, vrows_v, urows_v, out_v, sem):
        wid = lax.axis_index("s") * _NC + lax.axis_index("c")
        wbase = wid * per_w

        def chunk_body(ci, carry):
            gbase = pl.multiple_of(wbase + ci * _CB, _CB)
            pltpu.sync_copy(center_hbm.at[pl.ds(gbase, _CB)], cen_v)
            pltpu.sync_copy(ctx_hbm.at[pl.ds(gbase, _CB)], ctx_v)
            copies = [pltpu.async_copy(ev_hbm.at[cen_v], vrows_v, sem)]
            for b in range(_CB):
                copies.append(pltpu.async_copy(
                    eu_hbm.at[ctx_v.at[b]],
                    urows_v.at[pl.ds(b * L, L)], sem))
            for cp in copies:
                cp.wait()
            lanes = lax.iota(jnp.int32, 16)
            perms = [lanes ^ s for s in (8, 4, 2, 1)]
            group_counts = [16] * (L // 16) + (
                [L % 16] if L % 16 else [])

            def b_body(b, c):
                vk = [vrows_v[b, pl.ds(k * 16, 16)] for k in range(nk)]
                obase = b * L
                for g, cnt in enumerate(group_counts):
                    r = jnp.zeros((16,), jnp.float32)
                    for j in range(cnt):
                        row = obase + g * 16 + j
                        acc = urows_v[row, pl.ds(0, 16)] * vk[0]
                        for k in range(1, nk):
                            acc = acc + urows_v[row, pl.ds(k * 16, 16)] * vk[k]
                        for p in perms:
                            acc = acc + acc.at[p].get(
                                mode="promise_in_bounds")
                        r = jnp.where(lanes == j, acc, r)
                    out_v[pl.ds(obase + g * 16, 16)] = r
                return c

            lax.fori_loop(0, _CB, b_body, 0)
            pltpu.sync_copy(
                out_v.at[pl.ds(0, rows_per_chunk)],
                out_hbm.at[pl.ds(gbase * L, rows_per_chunk)])
            return carry

        lax.fori_loop(0, n_chunks, chunk_body, 0)

    return sc_kernel


def kernel(center, contexts_and_negatives, embed_v, embed_u):
    B, L = contexts_and_negatives.shape
    V, D = embed_v.shape
    center_flat = center.reshape(B).astype(jnp.int32)
    ctx = contexts_and_negatives.astype(jnp.int32)
    out = _make_sc_kernel(B, L, D, V)(ctx, center_flat, embed_v, embed_u)
    return out.reshape(B, 1, L)


# speedup vs baseline: 13.5926x; 1.2636x over previous
import functools

import jax
import jax.numpy as jnp
from jax import lax
from jax.experimental import pallas as pl
from jax.experimental.pallas import tpu as pltpu
from jax.experimental.pallas import tpu_sc as plsc

_INFO = plsc.get_sparse_core_info()
_NC = _INFO.num_cores
_NS = _INFO.num_subcores
_NW = _NC * _NS

_CB = 16


@functools.lru_cache(maxsize=None)
def _make_sc_kernel(B, L, D, V):
    del V
    per_w = B // _NW
    n_chunks = per_w // _CB
    rows_per_chunk = _CB * L
    nk = D // 16

    mesh = plsc.VectorSubcoreMesh(core_axis_name="c", subcore_axis_name="s")

    @functools.partial(
        pl.kernel,
        mesh=mesh,
        compiler_params=pltpu.CompilerParams(use_tc_tiling_on_sc=False),
        out_type=jax.ShapeDtypeStruct((B * L,), jnp.float32),
        scratch_types=[
            pltpu.VMEM((_CB, L), jnp.int32),
            pltpu.VMEM((_CB, L), jnp.int32),
            pltpu.VMEM((_CB,), jnp.int32),
            pltpu.VMEM((_CB,), jnp.int32),
            pltpu.VMEM((_CB, D), jnp.float32),
            pltpu.VMEM((_CB, D), jnp.float32),
            pltpu.VMEM((rows_per_chunk, D), jnp.float32),
            pltpu.VMEM((rows_per_chunk, D), jnp.float32),
            pltpu.VMEM((rows_per_chunk + 16,), jnp.float32),
            pltpu.SemaphoreType.DMA,
            pltpu.SemaphoreType.DMA,
        ],
    )
    def sc_kernel(ctx_hbm, center_hbm, ev_hbm, eu_hbm, out_hbm,
                  ctx0, ctx1, cen0, cen1, vr0, vr1, ur0, ur1, out_v,
                  sem0, sem1):
        ctxs, cens, vrs, urs, gsem = (
            [ctx0, ctx1], [cen0, cen1], [vr0, vr1], [ur0, ur1],
            [sem0, sem1])
        wid = lax.axis_index("s") * _NC + lax.axis_index("c")
        wbase = wid * per_w

        lanes = lax.iota(jnp.int32, 16)
        perm_tbl = {st: lanes ^ st for st in (8, 4, 2, 1)}
        mask_tbl = {st: (lanes & st) == 0 for st in (8, 4, 2, 1)}

        def issue(ci, s):
            gbase = pl.multiple_of(wbase + ci * _CB, _CB)
            pltpu.sync_copy(center_hbm.at[pl.ds(gbase, _CB)], cens[s])
            pltpu.sync_copy(ctx_hbm.at[pl.ds(gbase, _CB)], ctxs[s])
            pltpu.async_copy(ev_hbm.at[cens[s]], vrs[s], gsem[s])
            for b in range(_CB):
                pltpu.async_copy(
                    eu_hbm.at[ctxs[s].at[b]],
                    urs[s].at[pl.ds(b * L, L)], gsem[s])

        def drain(s):
            pltpu.make_async_copy(
                eu_hbm.at[pl.ds(0, _CB)], vrs[s], gsem[s]).wait()
            pltpu.make_async_copy(
                eu_hbm.at[pl.ds(0, rows_per_chunk)], urs[s], gsem[s]).wait()

        def compute(ci, s):
            gbase = pl.multiple_of(wbase + ci * _CB, _CB)
            ur = urs[s]
            vr = vrs[s]
            n_full = L // 16
            tail = L % 16

            def b_body(b, c):
                vk = [vr[b, pl.ds(k * 16, 16)] for k in range(nk)]
                obase = b * L

                def dot_acc(row):
                    a = ur[row, pl.ds(0, 16)] * vk[0]
                    for k in range(1, nk):
                        a = a + ur[row, pl.ds(k * 16, 16)] * vk[k]
                    return a

                for g in range(n_full):
                    accs = [dot_acc(obase + g * 16 + j) for j in range(16)]
                    for st in (8, 4, 2, 1):
                        half = len(accs) // 2
                        m = mask_tbl[st]
                        p = perm_tbl[st]
                        accs = [
                            jnp.where(
                                m, accs[j],
                                accs[j + half].at[p].get(
                                    mode="promise_in_bounds"))
                            + jnp.where(
                                m,
                                accs[j].at[p].get(mode="promise_in_bounds"),
                                accs[j + half])
                            for j in range(half)
                        ]
                    out_v[pl.ds(obase + g * 16, 16)] = accs[0]

                if tail:
                    r = jnp.zeros((16,), jnp.float32)
                    for j in range(tail):
                        acc = dot_acc(obase + n_full * 16 + j)
                        for st in (8, 4, 2, 1):
                            acc = acc + acc.at[perm_tbl[st]].get(
                                mode="promise_in_bounds")
                        r = jnp.where(lanes == j, acc, r)
                    out_v[pl.ds(obase + n_full * 16, 16)] = r
                return c

            lax.fori_loop(0, _CB, b_body, 0)
            pltpu.sync_copy(
                out_v.at[pl.ds(0, rows_per_chunk)],
                out_hbm.at[pl.ds(gbase * L, rows_per_chunk)])

        issue(0, 0)

        def pair_body(i, c):
            ci = i * 2
            issue(ci + 1, 1)
            drain(0)
            compute(ci, 0)

            @pl.when(ci + 2 < n_chunks)
            def _():
                issue(ci + 2, 0)

            drain(1)
            compute(ci + 1, 1)
            return c

        lax.fori_loop(0, n_chunks // 2, pair_body, 0)

    return sc_kernel


def kernel(center, contexts_and_negatives, embed_v, embed_u):
    B, L = contexts_and_negatives.shape
    V, D = embed_v.shape
    center_flat = center.reshape(B).astype(jnp.int32)
    ctx = contexts_and_negatives.astype(jnp.int32)
    out = _make_sc_kernel(B, L, D, V)(ctx, center_flat, embed_v, embed_u)
    return out.reshape(B, 1, L)


# speedup vs baseline: 13.7272x; 1.0099x over previous
import functools

import jax
import jax.numpy as jnp
from jax import lax
from jax.experimental import pallas as pl
from jax.experimental.pallas import tpu as pltpu
from jax.experimental.pallas import tpu_sc as plsc

_INFO = plsc.get_sparse_core_info()
_NC = _INFO.num_cores
_NS = _INFO.num_subcores
_NW = _NC * _NS

_CB = 16


@functools.lru_cache(maxsize=None)
def _make_sc_kernel(B, L, D, V):
    del V
    per_w = B // _NW
    n_chunks = per_w // _CB
    rows_per_chunk = _CB * L
    nk = D // 16

    mesh = plsc.VectorSubcoreMesh(core_axis_name="c", subcore_axis_name="s")

    @functools.partial(
        pl.kernel,
        mesh=mesh,
        compiler_params=pltpu.CompilerParams(use_tc_tiling_on_sc=False),
        out_type=jax.ShapeDtypeStruct((B * L,), jnp.float32),
        scratch_types=[
            pltpu.VMEM((_CB, L), jnp.int32),
            pltpu.VMEM((_CB, L), jnp.int32),
            pltpu.VMEM((_CB,), jnp.int32),
            pltpu.VMEM((_CB,), jnp.int32),
            pltpu.VMEM((_CB, D), jnp.float32),
            pltpu.VMEM((_CB, D), jnp.float32),
            pltpu.VMEM((rows_per_chunk, D), jnp.float32),
            pltpu.VMEM((rows_per_chunk, D), jnp.float32),
            pltpu.VMEM((rows_per_chunk + 16,), jnp.float32),
            pltpu.SemaphoreType.DMA,
            pltpu.SemaphoreType.DMA,
        ],
    )
    def sc_kernel(ctx_hbm, center_hbm, ev_hbm, eu_hbm, out_hbm,
                  ctx0, ctx1, cen0, cen1, vr0, vr1, ur0, ur1, out_v,
                  sem0, sem1):
        ctxs, cens, vrs, urs, gsem = (
            [ctx0, ctx1], [cen0, cen1], [vr0, vr1], [ur0, ur1],
            [sem0, sem1])
        wid = lax.axis_index("s") * _NC + lax.axis_index("c")
        wbase = wid * per_w

        lanes = lax.iota(jnp.int32, 16)
        perm_tbl = {st: lanes ^ st for st in (8, 4, 2, 1)}
        mask_tbl = {st: (lanes & st) == 0 for st in (8, 4, 2, 1)}

        def issue(ci, s):
            gbase = pl.multiple_of(wbase + ci * _CB, _CB)
            pltpu.sync_copy(center_hbm.at[pl.ds(gbase, _CB)], cens[s])
            pltpu.sync_copy(ctx_hbm.at[pl.ds(gbase, _CB)], ctxs[s])
            pltpu.async_copy(ev_hbm.at[cens[s]], vrs[s], gsem[s])
            for b in range(_CB):
                pltpu.async_copy(
                    eu_hbm.at[ctxs[s].at[b]],
                    urs[s].at[pl.ds(b * L, L)], gsem[s])

        def drain(s):
            pltpu.make_async_copy(
                eu_hbm.at[pl.ds(0, _CB)], vrs[s], gsem[s]).wait()
            pltpu.make_async_copy(
                eu_hbm.at[pl.ds(0, rows_per_chunk)], urs[s], gsem[s]).wait()

        def compute(ci, s):
            gbase = pl.multiple_of(wbase + ci * _CB, _CB)
            ur = urs[s]
            vr = vrs[s]
            n_full = L // 16
            tail = L % 16

            def dot_acc(row, vk):
                t = [ur[row, pl.ds(k * 16, 16)] * vk[k] for k in range(nk)]
                while len(t) > 1:
                    t = ([t[i] + t[i + 1] for i in range(0, len(t) - 1, 2)]
                         + ([t[-1]] if len(t) % 2 else []))
                return t[0]

            def xperm(a, p):
                return a.at[p].get(mode="promise_in_bounds")

            def row_work(b):
                vk = [vr[b, pl.ds(k * 16, 16)] for k in range(nk)]
                obase = b * L
                for g in range(n_full):
                    accs = [dot_acc(obase + g * 16 + j, vk)
                            for j in range(16)]
                    for st in (8, 4, 2, 1):
                        half = len(accs) // 2
                        m = mask_tbl[st]
                        p = perm_tbl[st]
                        accs = [
                            jnp.where(m, accs[j], accs[j + half])
                            + xperm(jnp.where(m, accs[j + half], accs[j]), p)
                            for j in range(half)
                        ]
                    out_v[pl.ds(obase + g * 16, 16)] = accs[0]

                if tail:
                    r = jnp.zeros((16,), jnp.float32)
                    for j in range(tail):
                        acc = dot_acc(obase + n_full * 16 + j, vk)
                        for st in (8, 4, 2, 1):
                            acc = acc + xperm(acc, perm_tbl[st])
                        r = jnp.where(lanes == j, acc, r)
                    out_v[pl.ds(obase + n_full * 16, 16)] = r

            def b_body(i, c):
                row_work(i * 2)
                row_work(i * 2 + 1)
                return c

            lax.fori_loop(0, _CB // 2, b_body, 0)
            pltpu.sync_copy(
                out_v.at[pl.ds(0, rows_per_chunk)],
                out_hbm.at[pl.ds(gbase * L, rows_per_chunk)])

        issue(0, 0)

        def pair_body(i, c):
            ci = i * 2
            issue(ci + 1, 1)
            drain(0)
            compute(ci, 0)

            @pl.when(ci + 2 < n_chunks)
            def _():
                issue(ci + 2, 0)

            drain(1)
            compute(ci + 1, 1)
            return c

        lax.fori_loop(0, n_chunks // 2, pair_body, 0)

    return sc_kernel


def kernel(center, contexts_and_negatives, embed_v, embed_u):
    B, L = contexts_and_negatives.shape
    V, D = embed_v.shape
    center_flat = center.reshape(B).astype(jnp.int32)
    ctx = contexts_and_negatives.astype(jnp.int32)
    out = _make_sc_kernel(B, L, D, V)(ctx, center_flat, embed_v, embed_u)
    return out.reshape(B, 1, L)


# speedup vs baseline: 15.3341x; 1.1171x over previous
import functools

import jax
import jax.numpy as jnp
from jax import lax
from jax.experimental import pallas as pl
from jax.experimental.pallas import tpu as pltpu
from jax.experimental.pallas import tpu_sc as plsc

_INFO = plsc.get_sparse_core_info()
_NC = _INFO.num_cores
_NS = _INFO.num_subcores
_NW = _NC * _NS

_CB = 16


@functools.lru_cache(maxsize=None)
def _make_sc_kernel(B, L, D, V):
    del V
    per_w = B // _NW
    n_chunks = per_w // _CB
    rows_per_chunk = _CB * L
    nk = D // 16

    mesh = plsc.VectorSubcoreMesh(core_axis_name="c", subcore_axis_name="s")

    @functools.partial(
        pl.kernel,
        mesh=mesh,
        compiler_params=pltpu.CompilerParams(use_tc_tiling_on_sc=False),
        out_type=jax.ShapeDtypeStruct((B * L,), jnp.float32),
        scratch_types=[
            pltpu.VMEM((_CB, L), jnp.int32),
            pltpu.VMEM((_CB, L), jnp.int32),
            pltpu.VMEM((_CB,), jnp.int32),
            pltpu.VMEM((_CB,), jnp.int32),
            pltpu.VMEM((_CB, D), jnp.float32),
            pltpu.VMEM((_CB, D), jnp.float32),
            pltpu.VMEM((rows_per_chunk, D), jnp.float32),
            pltpu.VMEM((rows_per_chunk, D), jnp.float32),
            pltpu.VMEM((rows_per_chunk + 16,), jnp.float32),
            pltpu.VMEM((rows_per_chunk + 16,), jnp.float32),
            pltpu.SemaphoreType.DMA,
            pltpu.SemaphoreType.DMA,
            pltpu.SemaphoreType.DMA,
            pltpu.SemaphoreType.DMA,
            pltpu.SemaphoreType.DMA,
            pltpu.SemaphoreType.DMA,
        ],
    )
    def sc_kernel(ctx_hbm, center_hbm, ev_hbm, eu_hbm, out_hbm,
                  ctx0, ctx1, cen0, cen1, vr0, vr1, ur0, ur1, ov0, ov1,
                  sem0, sem1, isem0, isem1, osem0, osem1):
        ctxs, cens, vrs, urs, ovs = (
            [ctx0, ctx1], [cen0, cen1], [vr0, vr1], [ur0, ur1], [ov0, ov1])
        gsem, isem, osem = [sem0, sem1], [isem0, isem1], [osem0, osem1]
        wid = lax.axis_index("s") * _NC + lax.axis_index("c")
        wbase = wid * per_w

        lanes = lax.iota(jnp.int32, 16)
        perm_tbl = {st: lanes ^ st for st in (8, 4, 2, 1)}
        mask_tbl = {st: (lanes & st) == 0 for st in (8, 4, 2, 1)}

        def fetch_idx(ci, s):
            gbase = pl.multiple_of(wbase + ci * _CB, _CB)
            pltpu.async_copy(
                center_hbm.at[pl.ds(gbase, _CB)], cens[s], isem[s])
            pltpu.async_copy(ctx_hbm.at[pl.ds(gbase, _CB)], ctxs[s], isem[s])

        def wait_idx(s):
            pltpu.make_async_copy(
                center_hbm.at[pl.ds(0, _CB)], cens[s], isem[s]).wait()
            pltpu.make_async_copy(
                ctx_hbm.at[pl.ds(0, _CB)], ctxs[s], isem[s]).wait()

        def issue(s):
            pltpu.async_copy(ev_hbm.at[cens[s]], vrs[s], gsem[s])
            for b in range(_CB):
                pltpu.async_copy(
                    eu_hbm.at[ctxs[s].at[b]],
                    urs[s].at[pl.ds(b * L, L)], gsem[s])

        def drain(s):
            pltpu.make_async_copy(
                eu_hbm.at[pl.ds(0, _CB)], vrs[s], gsem[s]).wait()
            pltpu.make_async_copy(
                eu_hbm.at[pl.ds(0, rows_per_chunk)], urs[s], gsem[s]).wait()

        def drain_out(s):
            pltpu.make_async_copy(
                ovs[s].at[pl.ds(0, rows_per_chunk)],
                out_hbm.at[pl.ds(0, rows_per_chunk)], osem[s]).wait()

        def compute(ci, s):
            gbase = pl.multiple_of(wbase + ci * _CB, _CB)
            ur = urs[s]
            vr = vrs[s]
            out_v = ovs[s]
            n_full = L // 16
            tail = L % 16

            def dot_acc(row, vk):
                t = [ur[row, pl.ds(k * 16, 16)] * vk[k] for k in range(nk)]
                while len(t) > 1:
                    t = ([t[i] + t[i + 1] for i in range(0, len(t) - 1, 2)]
                         + ([t[-1]] if len(t) % 2 else []))
                return t[0]

            def xperm(a, p):
                return a.at[p].get(mode="promise_in_bounds")

            def row_work(b):
                vk = [vr[b, pl.ds(k * 16, 16)] for k in range(nk)]
                obase = b * L
                for g in range(n_full):
                    accs = [dot_acc(obase + g * 16 + j, vk)
                            for j in range(16)]
                    for st in (8, 4, 2, 1):
                        half = len(accs) // 2
                        m = mask_tbl[st]
                        p = perm_tbl[st]
                        accs = [
                            jnp.where(m, accs[j], accs[j + half])
                            + xperm(jnp.where(m, accs[j + half], accs[j]), p)
                            for j in range(half)
                        ]
                    out_v[pl.ds(obase + g * 16, 16)] = accs[0]

                if tail:
                    r = jnp.zeros((16,), jnp.float32)
                    for j in range(tail):
                        acc = dot_acc(obase + n_full * 16 + j, vk)
                        for st in (8, 4, 2, 1):
                            acc = acc + xperm(acc, perm_tbl[st])
                        r = jnp.where(lanes == j, acc, r)
                    out_v[pl.ds(obase + n_full * 16, 16)] = r

            def b_body(i, c):
                row_work(i * 2)
                row_work(i * 2 + 1)
                return c

            lax.fori_loop(0, _CB // 2, b_body, 0)
            pltpu.async_copy(
                out_v.at[pl.ds(0, rows_per_chunk)],
                out_hbm.at[pl.ds(gbase * L, rows_per_chunk)], osem[s])

        fetch_idx(0, 0)
        fetch_idx(1, 1)
        wait_idx(0)
        issue(0)

        def pair_body(i, c):
            ci = i * 2
            wait_idx(1)
            issue(1)

            drain(0)

            @pl.when(ci + 2 < n_chunks)
            def _():
                fetch_idx(ci + 2, 0)

            @pl.when(i > 0)
            def _():
                drain_out(0)

            compute(ci, 0)

            @pl.when(ci + 2 < n_chunks)
            def _():
                wait_idx(0)
                issue(0)

            drain(1)

            @pl.when(ci + 3 < n_chunks)
            def _():
                fetch_idx(ci + 3, 1)

            @pl.when(i > 0)
            def _():
                drain_out(1)

            compute(ci + 1, 1)
            return c

        lax.fori_loop(0, n_chunks // 2, pair_body, 0)
        drain_out(0)
        drain_out(1)

    return sc_kernel


def kernel(center, contexts_and_negatives, embed_v, embed_u):
    B, L = contexts_and_negatives.shape
    V, D = embed_v.shape
    center_flat = center.reshape(B).astype(jnp.int32)
    ctx = contexts_and_negatives.astype(jnp.int32)
    out = _make_sc_kernel(B, L, D, V)(ctx, center_flat, embed_v, embed_u)
    return out.reshape(B, 1, L)
